# baseline (speedup 1.0000x reference)
import functools

import jax
import jax.numpy as jnp
from jax import lax
from jax.experimental import pallas as pl
from jax.experimental.pallas import tpu as pltpu
from jax.experimental.pallas import tpu_sc as plsc

ALPHA = 0.2
CH = 128
NSUB = 16
NCORE = 2


def _elu(v):
    return jnp.where(v > 0, v, jnp.exp(v) - 1.0)


def _l2n(v):
    n = jnp.sqrt(jnp.sum(v * v, axis=1, keepdims=True))
    return v / jnp.maximum(n, 1e-12)


def _pre_nodes_body(ent_ref, asd_ref, s4_ref, went_ref,
                    x0_ref, p4_ref, entup_ref, s_ref):
    x = ent_ref[...]
    x0 = _l2n(x)
    p4 = jnp.dot(x0, asd_ref[...], preferred_element_type=jnp.float32)
    x0_ref[...] = x0
    p4_ref[...] = p4
    entup_ref[...] = jnp.dot(x0, went_ref[...],
                             preferred_element_type=jnp.float32)
    s_ref[...] = jnp.dot(p4, s4_ref[...], preferred_element_type=jnp.float32)


def _rel_body(rel_ref, art_ref, w_ref, aort_ref, a20t_ref, a21t_ref, a2ot_ref,
              pr01_ref, outrel_ref, pro_ref, sr01_ref, sro_ref):
    rel = rel_ref[...]
    pr01 = jnp.dot(rel, art_ref[...], preferred_element_type=jnp.float32)
    outrel = jnp.dot(rel, w_ref[...], preferred_element_type=jnp.float32)
    pro = jnp.dot(outrel, aort_ref[...], preferred_element_type=jnp.float32)
    pr01_ref[...] = pr01
    outrel_ref[...] = outrel
    pro_ref[...] = pro
    sr0 = jnp.dot(pr01[:, :64], a20t_ref[...],
                  preferred_element_type=jnp.float32)
    sr1 = jnp.dot(pr01[:, 64:], a21t_ref[...],
                  preferred_element_type=jnp.float32)
    sr01_ref[...] = jnp.concatenate([sr0, sr1], axis=1)
    sro_ref[...] = jnp.dot(pro, a2ot_ref[...],
                           preferred_element_type=jnp.float32)


def _mid_body(p4_ref, acc_ref, rs_ref, aot_ref, so2_ref, po_ref, so_ref):
    p4 = p4_ref[...]
    acc = acc_ref[...]
    rs = rs_ref[...]
    rs0 = rs[:, 0:1]
    rs1 = rs[:, 1:2]
    h0 = jnp.where(rs0 > 0,
                   p4[:, 0:64] + acc[:, 0:64] / jnp.maximum(rs0, 1e-30), 0.0)
    h1 = jnp.where(rs1 > 0,
                   p4[:, 128:192] + acc[:, 64:128] / jnp.maximum(rs1, 1e-30),
                   0.0)
    x = _elu(jnp.concatenate([h0, h1], axis=1))
    po = jnp.dot(x, aot_ref[...], preferred_element_type=jnp.float32)
    po_ref[...] = po
    so_ref[...] = jnp.dot(po, so2_ref[...], preferred_element_type=jnp.float32)


def _final_body(entup_ref, po_ref, acc_ref, rso_ref, msk_ref, out_ref):
    rso = rso_ref[...]
    h = jnp.where(rso > 0,
                  po_ref[:, 0:128] + acc_ref[...] / jnp.maximum(rso, 1e-30),
                  0.0)
    x2 = _elu(h)
    m = jnp.minimum(msk_ref[...], 1.0)
    o = entup_ref[...] + m * x2
    out_ref[...] = _l2n(o)


def _sc_body(with_mask, npad, rpad, ept, nrow,
             *refs):
    if with_mask:
        (edges, ss, sd, sr, pd_t, pr_t, zrow, zvec, bidx,
         acc_out, rs_out, msk_out,
         buf3, src_c, dst_c, typ_c, w_c, pd_rows, pr_tile,
         stab_s, stab_d, stab_r, sem_e, sem_g1, sem_s, sem_r,
         acc_sh, rs_sh, msk_sh) = refs
    else:
        (edges, ss, sd, sr, pd_t, pr_t, zrow, zvec,
         acc_out, rs_out,
         buf3, src_c, dst_c, typ_c, w_c, pd_rows, pr_tile,
         stab_s, stab_d, stab_r, sem_e, sem_g1, sem_s, sem_r,
         acc_sh, rs_sh) = refs

    c = lax.axis_index("c")
    s = lax.axis_index("s")

    pltpu.sync_copy(zrow, acc_sh.at[pl.ds(s * nrow, nrow)])

    @pl.when(s == 0)
    def _zero_vecs():
        pltpu.sync_copy(zvec, rs_sh)
        if with_mask:
            pltpu.sync_copy(zvec, msk_sh)

    pltpu.sync_copy(ss.at[c], stab_s)
    pltpu.sync_copy(sd.at[c], stab_d)
    pltpu.sync_copy(sr.at[c], stab_r)

    pltpu.sync_copy(pr_t.at[c], pr_tile)

    plsc.subcore_barrier()

    nchunk = ept // CH

    def issue_edges(i, b):
        base = (s * ept + i * CH) * 3
        return pltpu.async_copy(edges.at[pl.ds(base, CH * 3)], buf3.at[b],
                                sem_e)

    def wait_edges(b):
        pltpu.make_async_copy(edges.at[pl.ds(0, CH * 3)], buf3.at[b],
                              sem_e).wait()

    def compute_w(b):
        bvec = jnp.full((16,), b, jnp.int32)
        for g in range(CH // 16):
            i3 = (lax.iota(jnp.int32, 16) + g * 16) * 3
            vsrc = plsc.load_gather(buf3, [bvec, i3])
            vdst = plsc.load_gather(buf3, [bvec, i3 + 1])
            vtyp = plsc.load_gather(buf3, [bvec, i3 + 2])
            z16 = jnp.zeros((16,), jnp.int32)
            t = (plsc.load_gather(stab_s, [z16, vsrc])
                 + plsc.load_gather(stab_d, [z16, vdst])
                 + plsc.load_gather(stab_r, [z16, vtyp]))
            w = jnp.exp(jnp.minimum(-t, -ALPHA * t))
            sl = pl.ds(g * 16, 16)
            src_c[b, sl] = vsrc
            dst_c[b, sl] = vdst
            typ_c[b, sl] = vtyp
            w_c[b, sl] = w

    def issue_gathers(b):
        pltpu.async_copy(pd_t.at[c].at[dst_c.at[b]], pd_rows.at[b], sem_g1)

    def wait_gathers(b):
        pltpu.make_async_copy(pd_t.at[c].at[dst_c.at[b]], pd_rows.at[b],
                              sem_g1).wait()

    def scale(b):
        def g_body(g, carry2):
            w16 = w_c[b, pl.ds(g * 16, 16)]
            t16 = typ_c[b, pl.ds(g * 16, 16)]
            for k in range(16):
                wv = w16[k]
                tv = t16[k]
                e = g * 16 + k
                for j in range(4):
                    slj = pl.ds(j * 16, 16)
                    pd_rows[b, e, slj] = (pd_rows[b, e, slj]
                                          + pr_tile[tv, slj]) * wv
            return carry2

        lax.fori_loop(0, CH // 16, g_body, 0)

    def issue_scatters(b):
        pltpu.async_copy(pd_rows.at[b], acc_sh.at[src_c.at[b]], sem_s,
                         add=True)
        pltpu.async_copy(w_c.at[b], rs_sh.at[src_c.at[b]], sem_r, add=True)

    def wait_scatters(b):
        pltpu.make_async_copy(pd_rows.at[b], acc_sh.at[src_c.at[b]],
                              sem_s).wait()
        pltpu.make_async_copy(w_c.at[b], rs_sh.at[src_c.at[b]],
                              sem_r).wait()

    issue_edges(0, 0)
    wait_edges(0)
    compute_w(0)
    issue_gathers(0)
    issue_edges(1, 1)
    wait_edges(1)
    compute_w(1)
    issue_gathers(1)
    issue_edges(2, 0)
    wait_gathers(0)
    scale(0)
    issue_scatters(0)

    def body_static(i, b):
        b1 = 1 - b
        wait_edges(b)
        wait_scatters(b)
        compute_w(b)
        issue_gathers(b)

        @pl.when(i + 1 < nchunk)
        def _prefetch():
            issue_edges(i + 1, b1)

        wait_gathers(b1)
        scale(b1)
        issue_scatters(b1)

    def pair_body(t, carry):
        i = 2 * t + 2
        body_static(i, 0)
        body_static(i + 1, 1)
        return carry

    lax.fori_loop(0, (nchunk - 2) // 2, pair_body, 0)

    last = (nchunk - 1) % 2
    wait_gathers(last)
    scale(last)
    wait_scatters(1 - last)
    issue_scatters(last)
    wait_scatters(last)

    if with_mask:
        wid = c * NSUB + s
        pltpu.sync_copy(bidx.at[pl.ds(wid * CH, CH)], src_c.at[0])
        for g in range(CH // 16):
            w_c[0, pl.ds(g * 16, 16)] = jnp.full((16,), 1.0, jnp.float32)
        pltpu.sync_copy(w_c.at[0], msk_sh.at[src_c.at[0]], add=True)

    plsc.subcore_barrier()

    row0 = s * nrow
    pltpu.sync_copy(acc_sh.at[pl.ds(row0, nrow)],
                    acc_out.at[c, pl.ds(row0, nrow)])
    pltpu.sync_copy(rs_sh.at[pl.ds(row0, nrow)],
                    rs_out.at[c, 0, pl.ds(row0, nrow)])
    if with_mask:
        pltpu.sync_copy(msk_sh.at[pl.ds(row0, nrow)],
                        msk_out.at[c, 0, pl.ds(row0, nrow)])


def _make_sc_pass(with_mask, npad, rpad, ept):
    nrow = npad // NSUB
    mesh = plsc.VectorSubcoreMesh(core_axis_name="c", subcore_axis_name="s")
    out_type = [
        jax.ShapeDtypeStruct((NCORE, npad, 64), jnp.float32),
        jax.ShapeDtypeStruct((NCORE, 1, npad), jnp.float32),
    ]
    if with_mask:
        out_type.append(jax.ShapeDtypeStruct((NCORE, 1, npad), jnp.float32))
    scratch_types = [
        pltpu.VMEM((2, CH * 3), jnp.int32),
        pltpu.VMEM((2, CH), jnp.int32),
        pltpu.VMEM((2, CH), jnp.int32),
        pltpu.VMEM((2, CH), jnp.int32),
        pltpu.VMEM((2, CH), jnp.float32),
        pltpu.VMEM((2, CH, 64), jnp.float32),
        pltpu.VMEM((rpad, 64), jnp.float32),
        pltpu.VMEM((1, npad), jnp.float32),
        pltpu.VMEM((1, npad), jnp.float32),
        pltpu.VMEM((1, rpad), jnp.float32),
        pltpu.SemaphoreType.DMA,
        pltpu.SemaphoreType.DMA,
        pltpu.SemaphoreType.DMA,
        pltpu.SemaphoreType.DMA,
        pltpu.VMEM_SHARED((npad, 64), jnp.float32),
        pltpu.VMEM_SHARED((npad,), jnp.float32),
    ]
    if with_mask:
        scratch_types.append(pltpu.VMEM_SHARED((npad,), jnp.float32))
    body = functools.partial(_sc_body, with_mask, npad, rpad, ept, nrow)
    return pl.kernel(body, mesh=mesh, out_type=out_type,
                     scratch_types=scratch_types,
                     compiler_params=pltpu.CompilerParams(
                         needs_layout_passes=False,
                         use_tc_tiling_on_sc=False,
                         skip_device_barrier=True,
                         disable_bounds_checks=True,
                         disable_semaphore_checks=True))


def kernel(Corpus_, edge_list, edge_type, batch_inputs, train_indices_nhop,
           confidence, entity_rank, entity_embeddings, relation_embeddings,
           W, W_entities, a0, a2_0, a1, a2_1, a_out, a2_out):
    f32 = jnp.float32
    i32 = jnp.int32
    src = edge_list[0]
    dst = edge_list[1]
    typ = edge_type.astype(i32)
    E = src.shape[0]
    N = entity_embeddings.shape[0]
    R = relation_embeddings.shape[0]
    din = entity_embeddings.shape[1]
    out1 = W.shape[1]

    blk = NSUB * 128
    npad = -(-N // blk) * blk
    if npad == N:
        npad += blk
    rpad = ((R + 15) // 16) * 16
    ept = -(-E // (NSUB * 2 * CH)) * 2 * CH
    etot = ept * NSUB
    nrow = npad // NSUB

    pad_e = etot - E
    pad_src = N + (jnp.arange(pad_e, dtype=i32) % (npad - N))
    zpad = jnp.zeros((pad_e,), i32)
    packed = jnp.concatenate(
        [jnp.stack([src, dst, typ], axis=1),
         jnp.stack([pad_src, zpad, zpad], axis=1)], axis=0).reshape(-1)

    asd_t = jnp.concatenate([a0[:, :din].T, a0[:, din:2 * din].T,
                             a1[:, :din].T, a1[:, din:2 * din].T], axis=1)
    z64 = jnp.zeros((64,), f32)
    s4 = jnp.stack([
        jnp.concatenate([a2_0[0], z64, z64, z64]),
        jnp.concatenate([z64, a2_0[0], z64, z64]),
        jnp.concatenate([z64, z64, a2_1[0], z64]),
        jnp.concatenate([z64, z64, z64, a2_1[0]]),
    ], axis=1)
    ar_t = jnp.concatenate([a0[:, 2 * din:].T, a1[:, 2 * din:].T], axis=1)
    aor_t = a_out[:, 2 * out1:].T
    ao_t = jnp.concatenate([a_out[:, :out1].T, a_out[:, out1:2 * out1].T],
                           axis=1)
    z128 = jnp.zeros((128,), f32)
    so2 = jnp.stack([
        jnp.concatenate([a2_out[0], z128]),
        jnp.concatenate([z128, a2_out[0]]),
    ], axis=1)
    a20_t = a2_0.T
    a21_t = a2_1.T
    a2o_t = a2_out.T

    nb = 10
    br = N // nb
    x0, p4, entup, snode = pl.pallas_call(
        _pre_nodes_body,
        grid=(nb,),
        in_specs=[
            pl.BlockSpec((br, din), lambda i: (i, 0)),
            pl.BlockSpec((din, 256), lambda i: (0, 0)),
            pl.BlockSpec((256, 4), lambda i: (0, 0)),
            pl.BlockSpec((din, out1), lambda i: (0, 0)),
        ],
        out_specs=[
            pl.BlockSpec((br, din), lambda i: (i, 0)),
            pl.BlockSpec((br, 256), lambda i: (i, 0)),
            pl.BlockSpec((br, out1), lambda i: (i, 0)),
            pl.BlockSpec((br, 4), lambda i: (i, 0)),
        ],
        out_shape=[
            jax.ShapeDtypeStruct((N, din), f32),
            jax.ShapeDtypeStruct((N, 256), f32),
            jax.ShapeDtypeStruct((N, out1), f32),
            jax.ShapeDtypeStruct((N, 4), f32),
        ],
    )(entity_embeddings, asd_t, s4, W_entities)

    pr01, outrel, pro, sr01, sro = pl.pallas_call(
        _rel_body,
        out_shape=[
            jax.ShapeDtypeStruct((R, 128), f32),
            jax.ShapeDtypeStruct((R, out1), f32),
            jax.ShapeDtypeStruct((R, out1), f32),
            jax.ShapeDtypeStruct((R, 2), f32),
            jax.ShapeDtypeStruct((R, 1), f32),
        ],
    )(relation_embeddings, ar_t, W, aor_t, a20_t, a21_t, a2o_t)

    pd1 = jnp.zeros((NCORE, npad, 64), f32)
    pd1 = pd1.at[0, :N].set(p4[:, 64:128]).at[1, :N].set(p4[:, 192:256])
    pr1 = jnp.zeros((NCORE, rpad, 64), f32)
    pr1 = pr1.at[0, :R].set(pr01[:, :64]).at[1, :R].set(pr01[:, 64:])
    ss1 = jnp.zeros((NCORE, 1, npad), f32)
    ss1 = ss1.at[0, 0, :N].set(snode[:, 0]).at[1, 0, :N].set(snode[:, 2])
    sd1 = jnp.zeros((NCORE, 1, npad), f32)
    sd1 = sd1.at[0, 0, :N].set(snode[:, 1]).at[1, 0, :N].set(snode[:, 3])
    sr1 = jnp.zeros((NCORE, 1, rpad), f32)
    sr1 = sr1.at[0, 0, :R].set(sr01[:, 0]).at[1, 0, :R].set(sr01[:, 1])
    zrow = jnp.zeros((nrow, 64), f32)
    zvec = jnp.zeros((npad,), f32)

    sc1 = _make_sc_pass(False, npad, rpad, ept)
    acc1, rs1 = sc1(packed, ss1, sd1, sr1, pd1, pr1, zrow, zvec)

    acc_cat = jnp.concatenate([acc1[0, :N], acc1[1, :N]], axis=1)
    rs_cat = jnp.stack([rs1[0, 0, :N], rs1[1, 0, :N]], axis=1)
    po, so = pl.pallas_call(
        _mid_body,
        grid=(nb,),
        in_specs=[
            pl.BlockSpec((br, 256), lambda i: (i, 0)),
            pl.BlockSpec((br, 128), lambda i: (i, 0)),
            pl.BlockSpec((br, 2), lambda i: (i, 0)),
            pl.BlockSpec((128, 256), lambda i: (0, 0)),
            pl.BlockSpec((256, 2), lambda i: (0, 0)),
        ],
        out_specs=[
            pl.BlockSpec((br, 256), lambda i: (i, 0)),
            pl.BlockSpec((br, 2), lambda i: (i, 0)),
        ],
        out_shape=[
            jax.ShapeDtypeStruct((N, 256), f32),
            jax.ShapeDtypeStruct((N, 2), f32),
        ],
    )(p4, acc_cat, rs_cat, ao_t, so2)

    pd2 = jnp.zeros((NCORE, npad, 64), f32)
    pd2 = pd2.at[0, :N].set(po[:, 128:192]).at[1, :N].set(po[:, 192:256])
    pr2 = jnp.zeros((NCORE, rpad, 64), f32)
    pr2 = pr2.at[0, :R].set(pro[:, :64]).at[1, :R].set(pro[:, 64:])
    ss2 = jnp.zeros((NCORE, 1, npad), f32).at[:, 0, :N].set(so[:, 0])
    sd2 = jnp.zeros((NCORE, 1, npad), f32).at[:, 0, :N].set(so[:, 1])
    sr2 = jnp.zeros((NCORE, 1, rpad), f32).at[:, 0, :R].set(sro[:, 0])

    bsz = NCORE * NSUB * CH
    b = batch_inputs[:, 2].astype(i32)
    pad_b = (-b.shape[0]) % bsz
    if pad_b:
        bfill = N + (jnp.arange(pad_b, dtype=i32) % (npad - N))
        b = jnp.concatenate([b, bfill])

    sc2 = _make_sc_pass(True, npad, rpad, ept)
    acc2, rs2, mskc = sc2(packed, ss2, sd2, sr2, pd2, pr2, zrow, zvec, b)

    acc2_cat = jnp.concatenate([acc2[0, :N], acc2[1, :N]], axis=1)
    rso = rs2[0, 0, :N][:, None]
    mcnt = (mskc[0, 0, :N] + mskc[1, 0, :N])[:, None]
    out_entity = pl.pallas_call(
        _final_body,
        grid=(nb,),
        in_specs=[
            pl.BlockSpec((br, 128), lambda i: (i, 0)),
            pl.BlockSpec((br, 256), lambda i: (i, 0)),
            pl.BlockSpec((br, 128), lambda i: (i, 0)),
            pl.BlockSpec((br, 1), lambda i: (i, 0)),
            pl.BlockSpec((br, 1), lambda i: (i, 0)),
        ],
        out_specs=pl.BlockSpec((br, 128), lambda i: (i, 0)),
        out_shape=jax.ShapeDtypeStruct((N, 128), f32),
    )(entup, po, acc2_cat, rso, mcnt)

    return (out_entity, outrel)


# speedup vs baseline: 1.4443x; 1.4443x over previous
import functools

import jax
import jax.numpy as jnp
from jax import lax
from jax.experimental import pallas as pl
from jax.experimental.pallas import tpu as pltpu
from jax.experimental.pallas import tpu_sc as plsc

ALPHA = 0.2
CH = 128
NSUB = 16
NCORE = 2


def _elu(v):
    return jnp.where(v > 0, v, jnp.exp(v) - 1.0)


def _l2n(v):
    n = jnp.sqrt(jnp.sum(v * v, axis=1, keepdims=True))
    return v / jnp.maximum(n, 1e-12)


def _pre_nodes_body(ent_ref, asd_ref, s4_ref, went_ref,
                    x0_ref, p4_ref, entup_ref, s_ref):
    x = ent_ref[...]
    x0 = _l2n(x)
    p4 = jnp.dot(x0, asd_ref[...], preferred_element_type=jnp.float32)
    x0_ref[...] = x0
    p4_ref[...] = p4
    entup_ref[...] = jnp.dot(x0, went_ref[...],
                             preferred_element_type=jnp.float32)
    s_ref[...] = jnp.dot(p4, s4_ref[...], preferred_element_type=jnp.float32)


def _rel_body(rel_ref, art_ref, w_ref, aort_ref, a20t_ref, a21t_ref, a2ot_ref,
              pr01_ref, outrel_ref, pro_ref, sr01_ref, sro_ref):
    rel = rel_ref[...]
    pr01 = jnp.dot(rel, art_ref[...], preferred_element_type=jnp.float32)
    outrel = jnp.dot(rel, w_ref[...], preferred_element_type=jnp.float32)
    pro = jnp.dot(outrel, aort_ref[...], preferred_element_type=jnp.float32)
    pr01_ref[...] = pr01
    outrel_ref[...] = outrel
    pro_ref[...] = pro
    sr0 = jnp.dot(pr01[:, :64], a20t_ref[...],
                  preferred_element_type=jnp.float32)
    sr1 = jnp.dot(pr01[:, 64:], a21t_ref[...],
                  preferred_element_type=jnp.float32)
    sr01_ref[...] = jnp.concatenate([sr0, sr1], axis=1)
    sro_ref[...] = jnp.dot(pro, a2ot_ref[...],
                           preferred_element_type=jnp.float32)


def _mid_body(p4_ref, acc_ref, rs_ref, aot_ref, so2_ref, po_ref, so_ref):
    p4 = p4_ref[...]
    acc = acc_ref[...]
    rs = rs_ref[...]
    rs0 = rs[:, 0:1]
    rs1 = rs[:, 1:2]
    h0 = jnp.where(rs0 > 0,
                   p4[:, 0:64] + acc[:, 0:64] / jnp.maximum(rs0, 1e-30), 0.0)
    h1 = jnp.where(rs1 > 0,
                   p4[:, 128:192] + acc[:, 64:128] / jnp.maximum(rs1, 1e-30),
                   0.0)
    x = _elu(jnp.concatenate([h0, h1], axis=1))
    po = jnp.dot(x, aot_ref[...], preferred_element_type=jnp.float32)
    po_ref[...] = po
    so_ref[...] = jnp.dot(po, so2_ref[...], preferred_element_type=jnp.float32)


def _final_body(entup_ref, po_ref, acc_ref, rso_ref, msk_ref, out_ref):
    rso = rso_ref[...]
    h = jnp.where(rso > 0,
                  po_ref[:, 0:128] + acc_ref[...] / jnp.maximum(rso, 1e-30),
                  0.0)
    x2 = _elu(h)
    m = jnp.minimum(msk_ref[...], 1.0)
    o = entup_ref[...] + m * x2
    out_ref[...] = _l2n(o)


def _sc_body(with_mask, npad, rpad, ept, nrow,
             *refs):
    if with_mask:
        (edges, ss, sd, sr, pd_t, pr_t, zrow, zvec, bidx,
         acc_out, rs_out, msk_out,
         buf3, src_c, dst_c, typ_c, w_c, pd_rows, pr_rows,
         stab_s, stab_d, stab_r, sem_e, sem_g1, sem_g2, sem_s, sem_r,
         acc_sh, rs_sh, msk_sh) = refs
    else:
        (edges, ss, sd, sr, pd_t, pr_t, zrow, zvec,
         acc_out, rs_out,
         buf3, src_c, dst_c, typ_c, w_c, pd_rows, pr_rows,
         stab_s, stab_d, stab_r, sem_e, sem_g1, sem_g2, sem_s, sem_r,
         acc_sh, rs_sh) = refs

    c = lax.axis_index("c")
    s = lax.axis_index("s")

    pltpu.sync_copy(zrow, acc_sh.at[pl.ds(s * nrow, nrow)])

    @pl.when(s == 0)
    def _zero_vecs():
        pltpu.sync_copy(zvec, rs_sh)
        if with_mask:
            pltpu.sync_copy(zvec, msk_sh)

    pltpu.sync_copy(ss.at[c], stab_s)
    pltpu.sync_copy(sd.at[c], stab_d)
    pltpu.sync_copy(sr.at[c], stab_r)

    plsc.subcore_barrier()

    nchunk = ept // CH

    def issue_edges(i, b):
        base = (s * ept + i * CH) * 3
        return pltpu.async_copy(edges.at[pl.ds(base, CH * 3)], buf3.at[b],
                                sem_e)

    def wait_edges(b):
        pltpu.make_async_copy(edges.at[pl.ds(0, CH * 3)], buf3.at[b],
                              sem_e).wait()

    def compute_w(b):
        z16 = jnp.zeros((16,), jnp.int32)
        for g in range(CH // 16):
            sl = pl.ds(g * 16, 16)
            vsrc = buf3[b, sl]
            vdst = buf3[b, pl.ds(CH + g * 16, 16)]
            vtyp = buf3[b, pl.ds(2 * CH + g * 16, 16)]
            t = (plsc.load_gather(stab_s, [z16, vsrc])
                 + plsc.load_gather(stab_d, [z16, vdst])
                 + plsc.load_gather(stab_r, [z16, vtyp]))
            w = jnp.exp(jnp.minimum(-t, -ALPHA * t))
            src_c[b, sl] = vsrc
            dst_c[b, sl] = vdst
            typ_c[b, sl] = vtyp
            w_c[b, sl] = w

    def issue_gathers(b):
        pltpu.async_copy(pd_t.at[c].at[dst_c.at[b]], pd_rows.at[b], sem_g1)
        pltpu.async_copy(pr_t.at[c].at[typ_c.at[b]], pr_rows.at[b], sem_g2)

    def wait_gathers(b):
        pltpu.make_async_copy(pd_t.at[c].at[dst_c.at[b]], pd_rows.at[b],
                              sem_g1).wait()
        pltpu.make_async_copy(pr_t.at[c].at[typ_c.at[b]], pr_rows.at[b],
                              sem_g2).wait()

    def scale(b):
        def g_body(g, carry2):
            w16 = w_c[b, pl.ds(g * 16, 16)]
            for k in range(16):
                wv = w16[k]
                e = g * 16 + k
                for j in range(4):
                    slj = pl.ds(j * 16, 16)
                    pd_rows[b, e, slj] = (pd_rows[b, e, slj]
                                          + pr_rows[b, e, slj]) * wv
            return carry2

        lax.fori_loop(0, CH // 16, g_body, 0)

    def issue_scatters(b):
        pltpu.async_copy(pd_rows.at[b], acc_sh.at[src_c.at[b]], sem_s,
                         add=True)
        pltpu.async_copy(w_c.at[b], rs_sh.at[src_c.at[b]], sem_r, add=True)

    def wait_scatters(b):
        pltpu.make_async_copy(pd_rows.at[b], acc_sh.at[src_c.at[b]],
                              sem_s).wait()
        pltpu.make_async_copy(w_c.at[b], rs_sh.at[src_c.at[b]],
                              sem_r).wait()

    issue_edges(0, 0)
    wait_edges(0)
    compute_w(0)
    issue_gathers(0)
    issue_edges(1, 1)
    wait_edges(1)
    compute_w(1)
    issue_gathers(1)
    issue_edges(2, 0)
    wait_gathers(0)
    scale(0)
    issue_scatters(0)

    def body_static(i, b):
        b1 = 1 - b
        wait_edges(b)
        wait_scatters(b)
        compute_w(b)
        issue_gathers(b)

        @pl.when(i + 1 < nchunk)
        def _prefetch():
            issue_edges(i + 1, b1)

        wait_gathers(b1)
        scale(b1)
        issue_scatters(b1)

    def pair_body(t, carry):
        i = 2 * t + 2
        body_static(i, 0)
        body_static(i + 1, 1)
        return carry

    lax.fori_loop(0, (nchunk - 2) // 2, pair_body, 0)

    last = (nchunk - 1) % 2
    wait_gathers(last)
    scale(last)
    wait_scatters(1 - last)
    issue_scatters(last)
    wait_scatters(last)

    if with_mask:
        wid = c * NSUB + s
        pltpu.sync_copy(bidx.at[pl.ds(wid * CH, CH)], src_c.at[0])
        for g in range(CH // 16):
            w_c[0, pl.ds(g * 16, 16)] = jnp.full((16,), 1.0, jnp.float32)
        pltpu.sync_copy(w_c.at[0], msk_sh.at[src_c.at[0]], add=True)

    plsc.subcore_barrier()

    row0 = s * nrow
    pltpu.sync_copy(acc_sh.at[pl.ds(row0, nrow)],
                    acc_out.at[c, pl.ds(row0, nrow)])
    pltpu.sync_copy(rs_sh.at[pl.ds(row0, nrow)],
                    rs_out.at[c, 0, pl.ds(row0, nrow)])
    if with_mask:
        pltpu.sync_copy(msk_sh.at[pl.ds(row0, nrow)],
                        msk_out.at[c, 0, pl.ds(row0, nrow)])


def _make_sc_pass(with_mask, npad, rpad, ept):
    nrow = npad // NSUB
    mesh = plsc.VectorSubcoreMesh(core_axis_name="c", subcore_axis_name="s")
    out_type = [
        jax.ShapeDtypeStruct((NCORE, npad, 64), jnp.float32),
        jax.ShapeDtypeStruct((NCORE, 1, npad), jnp.float32),
    ]
    if with_mask:
        out_type.append(jax.ShapeDtypeStruct((NCORE, 1, npad), jnp.float32))
    scratch_types = [
        pltpu.VMEM((2, CH * 3), jnp.int32),
        pltpu.VMEM((2, CH), jnp.int32),
        pltpu.VMEM((2, CH), jnp.int32),
        pltpu.VMEM((2, CH), jnp.int32),
        pltpu.VMEM((2, CH), jnp.float32),
        pltpu.VMEM((2, CH, 64), jnp.float32),
        pltpu.VMEM((2, CH, 64), jnp.float32),
        pltpu.VMEM((1, npad), jnp.float32),
        pltpu.VMEM((1, npad), jnp.float32),
        pltpu.VMEM((1, rpad), jnp.float32),
        pltpu.SemaphoreType.DMA,
        pltpu.SemaphoreType.DMA,
        pltpu.SemaphoreType.DMA,
        pltpu.SemaphoreType.DMA,
        pltpu.SemaphoreType.DMA,
        pltpu.VMEM_SHARED((npad, 64), jnp.float32),
        pltpu.VMEM_SHARED((npad,), jnp.float32),
    ]
    if with_mask:
        scratch_types.append(pltpu.VMEM_SHARED((npad,), jnp.float32))
    body = functools.partial(_sc_body, with_mask, npad, rpad, ept, nrow)
    return pl.kernel(body, mesh=mesh, out_type=out_type,
                     scratch_types=scratch_types,
                     compiler_params=pltpu.CompilerParams(
                         needs_layout_passes=False,
                         use_tc_tiling_on_sc=False,
                         skip_device_barrier=True,
                         disable_bounds_checks=True,
                         disable_semaphore_checks=True))


def kernel(Corpus_, edge_list, edge_type, batch_inputs, train_indices_nhop,
           confidence, entity_rank, entity_embeddings, relation_embeddings,
           W, W_entities, a0, a2_0, a1, a2_1, a_out, a2_out):
    f32 = jnp.float32
    i32 = jnp.int32
    src = edge_list[0]
    dst = edge_list[1]
    typ = edge_type.astype(i32)
    E = src.shape[0]
    N = entity_embeddings.shape[0]
    R = relation_embeddings.shape[0]
    din = entity_embeddings.shape[1]
    out1 = W.shape[1]

    blk = NSUB * 128
    npad = -(-N // blk) * blk
    if npad == N:
        npad += blk
    rpad = ((R + 15) // 16) * 16
    ept = -(-E // (NSUB * 2 * CH)) * 2 * CH
    etot = ept * NSUB
    nrow = npad // NSUB

    pad_e = etot - E
    pad_src = N + (jnp.arange(pad_e, dtype=i32) % (npad - N))
    zpad = jnp.zeros((pad_e,), i32)
    packed = jnp.concatenate(
        [jnp.stack([src, dst, typ], axis=1),
         jnp.stack([pad_src, zpad, zpad], axis=1)], axis=0)
    packed = packed.reshape(etot // CH, CH, 3)
    packed = jnp.swapaxes(packed, 1, 2).reshape(-1)

    asd_t = jnp.concatenate([a0[:, :din].T, a0[:, din:2 * din].T,
                             a1[:, :din].T, a1[:, din:2 * din].T], axis=1)
    z64 = jnp.zeros((64,), f32)
    s4 = jnp.stack([
        jnp.concatenate([a2_0[0], z64, z64, z64]),
        jnp.concatenate([z64, a2_0[0], z64, z64]),
        jnp.concatenate([z64, z64, a2_1[0], z64]),
        jnp.concatenate([z64, z64, z64, a2_1[0]]),
    ], axis=1)
    ar_t = jnp.concatenate([a0[:, 2 * din:].T, a1[:, 2 * din:].T], axis=1)
    aor_t = a_out[:, 2 * out1:].T
    ao_t = jnp.concatenate([a_out[:, :out1].T, a_out[:, out1:2 * out1].T],
                           axis=1)
    z128 = jnp.zeros((128,), f32)
    so2 = jnp.stack([
        jnp.concatenate([a2_out[0], z128]),
        jnp.concatenate([z128, a2_out[0]]),
    ], axis=1)
    a20_t = a2_0.T
    a21_t = a2_1.T
    a2o_t = a2_out.T

    nb = 10
    br = N // nb
    x0, p4, entup, snode = pl.pallas_call(
        _pre_nodes_body,
        grid=(nb,),
        in_specs=[
            pl.BlockSpec((br, din), lambda i: (i, 0)),
            pl.BlockSpec((din, 256), lambda i: (0, 0)),
            pl.BlockSpec((256, 4), lambda i: (0, 0)),
            pl.BlockSpec((din, out1), lambda i: (0, 0)),
        ],
        out_specs=[
            pl.BlockSpec((br, din), lambda i: (i, 0)),
            pl.BlockSpec((br, 256), lambda i: (i, 0)),
            pl.BlockSpec((br, out1), lambda i: (i, 0)),
            pl.BlockSpec((br, 4), lambda i: (i, 0)),
        ],
        out_shape=[
            jax.ShapeDtypeStruct((N, din), f32),
            jax.ShapeDtypeStruct((N, 256), f32),
            jax.ShapeDtypeStruct((N, out1), f32),
            jax.ShapeDtypeStruct((N, 4), f32),
        ],
    )(entity_embeddings, asd_t, s4, W_entities)

    pr01, outrel, pro, sr01, sro = pl.pallas_call(
        _rel_body,
        out_shape=[
            jax.ShapeDtypeStruct((R, 128), f32),
            jax.ShapeDtypeStruct((R, out1), f32),
            jax.ShapeDtypeStruct((R, out1), f32),
            jax.ShapeDtypeStruct((R, 2), f32),
            jax.ShapeDtypeStruct((R, 1), f32),
        ],
    )(relation_embeddings, ar_t, W, aor_t, a20_t, a21_t, a2o_t)

    pd1 = jnp.zeros((NCORE, npad, 64), f32)
    pd1 = pd1.at[0, :N].set(p4[:, 64:128]).at[1, :N].set(p4[:, 192:256])
    pr1 = jnp.zeros((NCORE, rpad, 64), f32)
    pr1 = pr1.at[0, :R].set(pr01[:, :64]).at[1, :R].set(pr01[:, 64:])
    ss1 = jnp.zeros((NCORE, 1, npad), f32)
    ss1 = ss1.at[0, 0, :N].set(snode[:, 0]).at[1, 0, :N].set(snode[:, 2])
    sd1 = jnp.zeros((NCORE, 1, npad), f32)
    sd1 = sd1.at[0, 0, :N].set(snode[:, 1]).at[1, 0, :N].set(snode[:, 3])
    sr1 = jnp.zeros((NCORE, 1, rpad), f32)
    sr1 = sr1.at[0, 0, :R].set(sr01[:, 0]).at[1, 0, :R].set(sr01[:, 1])
    zrow = jnp.zeros((nrow, 64), f32)
    zvec = jnp.zeros((npad,), f32)

    sc1 = _make_sc_pass(False, npad, rpad, ept)
    acc1, rs1 = sc1(packed, ss1, sd1, sr1, pd1, pr1, zrow, zvec)

    acc_cat = jnp.concatenate([acc1[0, :N], acc1[1, :N]], axis=1)
    rs_cat = jnp.stack([rs1[0, 0, :N], rs1[1, 0, :N]], axis=1)
    po, so = pl.pallas_call(
        _mid_body,
        grid=(nb,),
        in_specs=[
            pl.BlockSpec((br, 256), lambda i: (i, 0)),
            pl.BlockSpec((br, 128), lambda i: (i, 0)),
            pl.BlockSpec((br, 2), lambda i: (i, 0)),
            pl.BlockSpec((128, 256), lambda i: (0, 0)),
            pl.BlockSpec((256, 2), lambda i: (0, 0)),
        ],
        out_specs=[
            pl.BlockSpec((br, 256), lambda i: (i, 0)),
            pl.BlockSpec((br, 2), lambda i: (i, 0)),
        ],
        out_shape=[
            jax.ShapeDtypeStruct((N, 256), f32),
            jax.ShapeDtypeStruct((N, 2), f32),
        ],
    )(p4, acc_cat, rs_cat, ao_t, so2)

    pd2 = jnp.zeros((NCORE, npad, 64), f32)
    pd2 = pd2.at[0, :N].set(po[:, 128:192]).at[1, :N].set(po[:, 192:256])
    pr2 = jnp.zeros((NCORE, rpad, 64), f32)
    pr2 = pr2.at[0, :R].set(pro[:, :64]).at[1, :R].set(pro[:, 64:])
    ss2 = jnp.zeros((NCORE, 1, npad), f32).at[:, 0, :N].set(so[:, 0])
    sd2 = jnp.zeros((NCORE, 1, npad), f32).at[:, 0, :N].set(so[:, 1])
    sr2 = jnp.zeros((NCORE, 1, rpad), f32).at[:, 0, :R].set(sro[:, 0])

    bsz = NCORE * NSUB * CH
    b = batch_inputs[:, 2].astype(i32)
    pad_b = (-b.shape[0]) % bsz
    if pad_b:
        bfill = N + (jnp.arange(pad_b, dtype=i32) % (npad - N))
        b = jnp.concatenate([b, bfill])

    sc2 = _make_sc_pass(True, npad, rpad, ept)
    acc2, rs2, mskc = sc2(packed, ss2, sd2, sr2, pd2, pr2, zrow, zvec, b)

    acc2_cat = jnp.concatenate([acc2[0, :N], acc2[1, :N]], axis=1)
    rso = rs2[0, 0, :N][:, None]
    mcnt = (mskc[0, 0, :N] + mskc[1, 0, :N])[:, None]
    out_entity = pl.pallas_call(
        _final_body,
        grid=(nb,),
        in_specs=[
            pl.BlockSpec((br, 128), lambda i: (i, 0)),
            pl.BlockSpec((br, 256), lambda i: (i, 0)),
            pl.BlockSpec((br, 128), lambda i: (i, 0)),
            pl.BlockSpec((br, 1), lambda i: (i, 0)),
            pl.BlockSpec((br, 1), lambda i: (i, 0)),
        ],
        out_specs=pl.BlockSpec((br, 128), lambda i: (i, 0)),
        out_shape=jax.ShapeDtypeStruct((N, 128), f32),
    )(entup, po, acc2_cat, rso, mcnt)

    return (out_entity, outrel)
